# baseline (speedup 1.0000x reference)
import jax
import jax.numpy as jnp
from jax import lax
from jax.experimental import pallas as pl
from jax.experimental.pallas import tpu as pltpu
from jax.experimental.pallas import tpu_sc as plsc

ALPHA = 0.5
POS_WEIGHT = 3.0
WEIGHT_DISTANCE = 0.15

NC = 2
NS = 16
L = 16

N_EDGES = 320000
C2 = N_EDGES // (NS * NC)


def _log1p_poly_z(z):
    z2 = z * z
    return 2.0 * z * (1.0 + z2 * (1.0 / 3.0 + z2 * (1.0 / 5.0 + z2 * (1.0 / 7.0))))


def _sc_bce_body(l_hbm, y_hbm, p_hbm, out_hbm,
                 l_v, y_v, p_v, st_v, gat_v, sh_b, sem_l, sem_y):
    core = lax.axis_index("c")
    sid = lax.axis_index("s")
    wid = sid * NC + core

    H = C2 // 2
    base = wid * C2
    cp_l0 = pltpu.async_copy(l_hbm.at[pl.ds(base, H)], l_v.at[pl.ds(0, H)], sem_l)
    cp_y0 = pltpu.async_copy(y_hbm.at[pl.ds(base, H)], y_v.at[pl.ds(0, H)], sem_y)
    cp_l1 = pltpu.async_copy(l_hbm.at[pl.ds(base + H, H)], l_v.at[pl.ds(H, H)], sem_l)
    cp_y1 = pltpu.async_copy(y_hbm.at[pl.ds(base + H, H)], y_v.at[pl.ds(H, H)], sem_y)
    cp_l0.wait()
    cp_y0.wait()

    zeros = jnp.zeros((L,), jnp.float32)

    def body(i, b_acc):
        lv = l_v[pl.ds(i, L)]
        yv = y_v[pl.ds(i, L)]
        al = jnp.abs(lv)
        t = jnp.exp(-al)
        u = 1.0 + t
        v2 = 2.0 + t
        r = 1.0 / (u * v2)
        w = v2 * r
        z = t * u * r
        p = jnp.where(lv >= 0.0, w, t * w)
        p_v[pl.ds(i, L)] = p
        sp = jnp.maximum(-lv, 0.0) + _log1p_poly_z(z)
        return b_acc + sp * (1.0 + 2.0 * yv) + (1.0 - yv) * lv

    b_acc = plsc.parallel_loop(0, H, step=L, unroll=8, carry=zeros)(body)
    cp_l1.wait()
    cp_y1.wait()
    b_acc = plsc.parallel_loop(H, C2, step=L, unroll=8, carry=b_acc)(body)
    pltpu.sync_copy(p_v, p_hbm.at[pl.ds(base, C2)])

    st_v[...] = b_acc
    pltpu.sync_copy(st_v, sh_b.at[pl.ds(sid * L, L)])
    plsc.subcore_barrier()

    @pl.when(sid == 0)
    def _():
        pltpu.sync_copy(sh_b, gat_v)
        bv = gat_v[pl.ds(0, L)]
        for j in range(1, NS):
            bv = bv + gat_v[pl.ds(j * L, L)]
        bsum = bv[0]
        for j in range(1, L):
            bsum = bsum + bv[j]
        lane = lax.iota(jnp.int32, L)
        st_v[...] = jnp.where(lane == 0, bsum, 0.0)
        pltpu.sync_copy(st_v, out_hbm.at[pl.ds(core * L, L)])


_sc_bce = pl.kernel(
    _sc_bce_body,
    out_type=(
        jax.ShapeDtypeStruct((N_EDGES,), jnp.float32),
        jax.ShapeDtypeStruct((NC * L,), jnp.float32),
    ),
    mesh=plsc.VectorSubcoreMesh(core_axis_name="c", subcore_axis_name="s"),
    scratch_types=[
        pltpu.VMEM((C2,), jnp.float32),
        pltpu.VMEM((C2,), jnp.float32),
        pltpu.VMEM((C2,), jnp.float32),
        pltpu.VMEM((L,), jnp.float32),
        pltpu.VMEM((NS * L,), jnp.float32),
        pltpu.VMEM_SHARED((NS * L,), jnp.float32),
        pltpu.SemaphoreType.DMA,
        pltpu.SemaphoreType.DMA,
    ],
)


def _tc_rank_body(d_ref, p_ref, out_ref):
    d = d_ref[...]
    maxd = jnp.max(d)
    mind = jnp.min(d)
    m = maxd + 1e-8
    min_s = WEIGHT_DISTANCE * (1.0 - maxd / m)
    max_s = WEIGHT_DISTANCE * (1.0 - mind / m)
    den = max_s - min_s + 1e-8
    coef = (WEIGHT_DISTANCE / m) / den
    inv = 1.0 - coef * (maxd - d)
    r = p_ref[...] - inv
    out_ref[...] = jnp.sum(r * r).reshape(1, 1)


_tc_rank = pl.pallas_call(
    _tc_rank_body,
    out_shape=jax.ShapeDtypeStruct((1, 1), jnp.float32),
    in_specs=[
        pl.BlockSpec(memory_space=pltpu.VMEM),
        pl.BlockSpec(memory_space=pltpu.VMEM),
    ],
    out_specs=pl.BlockSpec(memory_space=pltpu.VMEM),
)


def kernel(logits, x, edge_index, edge_attr, y):
    del x, edge_index
    p_flat, bce_out = _sc_bce(logits, y)
    d = jnp.reshape(edge_attr, (N_EDGES,))
    d2 = jnp.reshape(d, (2500, 128))
    p2 = jnp.reshape(p_flat, (2500, 128))
    rank = _tc_rank(d2, p2)
    bce_sum = bce_out[0] + bce_out[L]
    n = jnp.float32(N_EDGES)
    return (1.0 - ALPHA) * (bce_sum / n) + ALPHA * (rank[0, 0] / n)


# speedup vs baseline: 1.0030x; 1.0030x over previous
import jax
import jax.numpy as jnp
from jax import lax
from jax.experimental import pallas as pl
from jax.experimental.pallas import tpu as pltpu
from jax.experimental.pallas import tpu_sc as plsc

ALPHA = 0.5
POS_WEIGHT = 3.0
WEIGHT_DISTANCE = 0.15

NC = 2
NS = 16
L = 16

N_EDGES = 320000
C2 = N_EDGES // (NS * NC)


def _log1p_poly_z(z):
    z2 = z * z
    return 2.0 * z * (1.0 + z2 * (1.0 / 3.0 + z2 * (1.0 / 5.0 + z2 * (1.0 / 7.0))))


def _sc_bce_body(l_hbm, y_hbm, p_hbm, out_hbm,
                 l_v, y_v, p_v, st_v, gat_v, sh_b, sem_l, sem_y):
    core = lax.axis_index("c")
    sid = lax.axis_index("s")
    wid = sid * NC + core

    base = wid * C2
    cp_l = pltpu.async_copy(l_hbm.at[pl.ds(base, C2)], l_v, sem_l)
    cp_y = pltpu.async_copy(y_hbm.at[pl.ds(base, C2)], y_v, sem_y)
    cp_l.wait()
    cp_y.wait()

    zeros = jnp.zeros((L,), jnp.float32)

    def body(i, b_acc):
        lv = l_v[pl.ds(i, L)]
        yv = y_v[pl.ds(i, L)]
        al = jnp.abs(lv)
        t = jnp.exp(-al)
        u = 1.0 + t
        v2 = 2.0 + t
        r = 1.0 / (u * v2)
        w = v2 * r
        z = t * u * r
        p = jnp.where(lv >= 0.0, w, t * w)
        p_v[pl.ds(i, L)] = p
        sp = jnp.maximum(-lv, 0.0) + _log1p_poly_z(z)
        return b_acc + sp * (1.0 + 2.0 * yv) + (1.0 - yv) * lv

    b_acc = plsc.parallel_loop(0, C2, step=L, unroll=4, carry=zeros)(body)
    pltpu.sync_copy(p_v, p_hbm.at[pl.ds(base, C2)])

    st_v[...] = b_acc
    pltpu.sync_copy(st_v, sh_b.at[pl.ds(sid * L, L)])
    plsc.subcore_barrier()

    @pl.when(sid == 0)
    def _():
        pltpu.sync_copy(sh_b, gat_v)
        bv = gat_v[pl.ds(0, L)]
        for j in range(1, NS):
            bv = bv + gat_v[pl.ds(j * L, L)]
        bsum = bv[0]
        for j in range(1, L):
            bsum = bsum + bv[j]
        lane = lax.iota(jnp.int32, L)
        st_v[...] = jnp.where(lane == 0, bsum, 0.0)
        pltpu.sync_copy(st_v, out_hbm.at[pl.ds(core * L, L)])


_sc_bce = pl.kernel(
    _sc_bce_body,
    out_type=(
        jax.ShapeDtypeStruct((N_EDGES,), jnp.float32),
        jax.ShapeDtypeStruct((NC * L,), jnp.float32),
    ),
    mesh=plsc.VectorSubcoreMesh(core_axis_name="c", subcore_axis_name="s"),
    scratch_types=[
        pltpu.VMEM((C2,), jnp.float32),
        pltpu.VMEM((C2,), jnp.float32),
        pltpu.VMEM((C2,), jnp.float32),
        pltpu.VMEM((L,), jnp.float32),
        pltpu.VMEM((NS * L,), jnp.float32),
        pltpu.VMEM_SHARED((NS * L,), jnp.float32),
        pltpu.SemaphoreType.DMA,
        pltpu.SemaphoreType.DMA,
    ],
)


def _tc_rank_body(d_ref, p_ref, out_ref):
    d = d_ref[...]
    maxd = jnp.max(d)
    mind = jnp.min(d)
    m = maxd + 1e-8
    min_s = WEIGHT_DISTANCE * (1.0 - maxd / m)
    max_s = WEIGHT_DISTANCE * (1.0 - mind / m)
    den = max_s - min_s + 1e-8
    coef = (WEIGHT_DISTANCE / m) / den
    inv = 1.0 - coef * (maxd - d)
    r = p_ref[...] - inv
    out_ref[...] = jnp.sum(r * r).reshape(1, 1)


_tc_rank = pl.pallas_call(
    _tc_rank_body,
    out_shape=jax.ShapeDtypeStruct((1, 1), jnp.float32),
    in_specs=[
        pl.BlockSpec(memory_space=pltpu.VMEM),
        pl.BlockSpec(memory_space=pltpu.VMEM),
    ],
    out_specs=pl.BlockSpec(memory_space=pltpu.VMEM),
)


def kernel(logits, x, edge_index, edge_attr, y):
    del x, edge_index
    p_flat, bce_out = _sc_bce(logits, y)
    d = jnp.reshape(edge_attr, (N_EDGES,))
    d2 = jnp.reshape(d, (2500, 128))
    p2 = jnp.reshape(p_flat, (2500, 128))
    rank = _tc_rank(d2, p2)
    bce_sum = bce_out[0] + bce_out[L]
    n = jnp.float32(N_EDGES)
    return (1.0 - ALPHA) * (bce_sum / n) + ALPHA * (rank[0, 0] / n)


# speedup vs baseline: 1.0690x; 1.0658x over previous
import jax
import jax.numpy as jnp
from jax import lax
from jax.experimental import pallas as pl
from jax.experimental.pallas import tpu as pltpu
from jax.experimental.pallas import tpu_sc as plsc

ALPHA = 0.5
POS_WEIGHT = 3.0
WEIGHT_DISTANCE = 0.15

NC = 2
NS = 16
L = 16

N_EDGES = 320000
C2 = N_EDGES // (NS * NC)


def _log1p_poly_z(z):
    z2 = z * z
    return 2.0 * z * (1.0 + z2 * (1.0 / 3.0 + z2 * (1.0 / 5.0 + z2 * (1.0 / 7.0))))


def _sc_bce_body(l_hbm, y_hbm, p_hbm, out_hbm,
                 l_v, y_v, p_v, st_v, sem_l, sem_y):
    core = lax.axis_index("c")
    sid = lax.axis_index("s")
    wid = sid * NC + core

    base = wid * C2
    cp_l = pltpu.async_copy(l_hbm.at[pl.ds(base, C2)], l_v, sem_l)
    cp_y = pltpu.async_copy(y_hbm.at[pl.ds(base, C2)], y_v, sem_y)
    cp_l.wait()
    cp_y.wait()

    zeros = jnp.zeros((L,), jnp.float32)

    def body(i, b_acc):
        lv = l_v[pl.ds(i, L)]
        yv = y_v[pl.ds(i, L)]
        al = jnp.abs(lv)
        t = jnp.exp(-al)
        w = 1.0 / (1.0 + t)
        p = jnp.where(lv >= 0.0, w, t * w)
        p_v[pl.ds(i, L)] = p
        z = t / (2.0 + t)
        sp = jnp.maximum(-lv, 0.0) + _log1p_poly_z(z)
        return b_acc + sp * (1.0 + 2.0 * yv) + (1.0 - yv) * lv

    b_acc = plsc.parallel_loop(0, C2, step=L, unroll=4, carry=zeros)(body)
    pltpu.sync_copy(p_v, p_hbm.at[pl.ds(base, C2)])
    st_v[...] = b_acc
    pltpu.sync_copy(st_v, out_hbm.at[pl.ds(wid * L, L)])


_sc_bce = pl.kernel(
    _sc_bce_body,
    out_type=(
        jax.ShapeDtypeStruct((N_EDGES,), jnp.float32),
        jax.ShapeDtypeStruct((NC * NS * L,), jnp.float32),
    ),
    mesh=plsc.VectorSubcoreMesh(core_axis_name="c", subcore_axis_name="s"),
    scratch_types=[
        pltpu.VMEM((C2,), jnp.float32),
        pltpu.VMEM((C2,), jnp.float32),
        pltpu.VMEM((C2,), jnp.float32),
        pltpu.VMEM((L,), jnp.float32),
        pltpu.SemaphoreType.DMA,
        pltpu.SemaphoreType.DMA,
    ],
)


def _tc_rank_body(d_ref, p_ref, out_ref):
    d = d_ref[...]
    maxd = jnp.max(d)
    mind = jnp.min(d)
    m = maxd + 1e-8
    min_s = WEIGHT_DISTANCE * (1.0 - maxd / m)
    max_s = WEIGHT_DISTANCE * (1.0 - mind / m)
    den = max_s - min_s + 1e-8
    coef = (WEIGHT_DISTANCE / m) / den
    inv = 1.0 - coef * (maxd - d)
    r = p_ref[...] - inv
    out_ref[...] = jnp.sum(r * r).reshape(1, 1)


_tc_rank = pl.pallas_call(
    _tc_rank_body,
    out_shape=jax.ShapeDtypeStruct((1, 1), jnp.float32),
    in_specs=[
        pl.BlockSpec(memory_space=pltpu.VMEM),
        pl.BlockSpec(memory_space=pltpu.VMEM),
    ],
    out_specs=pl.BlockSpec(memory_space=pltpu.VMEM),
)


def kernel(logits, x, edge_index, edge_attr, y):
    del x, edge_index
    p_flat, bce_out = _sc_bce(logits, y)
    d = jnp.reshape(edge_attr, (N_EDGES,))
    d2 = jnp.reshape(d, (2500, 128))
    p2 = jnp.reshape(p_flat, (2500, 128))
    rank = _tc_rank(d2, p2)
    bce_sum = jnp.sum(bce_out)
    n = jnp.float32(N_EDGES)
    return (1.0 - ALPHA) * (bce_sum / n) + ALPHA * (rank[0, 0] / n)


# speedup vs baseline: 1.1601x; 1.0852x over previous
import jax
import jax.numpy as jnp
from jax import lax
from jax.experimental import pallas as pl
from jax.experimental.pallas import tpu as pltpu
from jax.experimental.pallas import tpu_sc as plsc

ALPHA = 0.5
POS_WEIGHT = 3.0
WEIGHT_DISTANCE = 0.15

NC = 2
NS = 16
L = 16

N_EDGES = 320000
C2 = N_EDGES // (NS * NC)


def _log1p_poly_z(z):
    z2 = z * z
    return 2.0 * z * (1.0 + z2 * (1.0 / 3.0 + z2 * (1.0 / 5.0 + z2 * (1.0 / 7.0))))


def _sc_bce_body(l_hbm, y_hbm, p_hbm, out_hbm,
                 l_v, y_v, p_v, st_v, sem_l, sem_y):
    core = lax.axis_index("c")
    sid = lax.axis_index("s")
    wid = sid * NC + core

    base = wid * C2
    cp_l = pltpu.async_copy(l_hbm.at[pl.ds(base, C2)], l_v, sem_l)
    cp_y = pltpu.async_copy(y_hbm.at[pl.ds(base, C2)], y_v, sem_y)
    cp_l.wait()
    cp_y.wait()

    zeros = jnp.zeros((L,), jnp.float32)

    def body(i, b_acc):
        lv = l_v[pl.ds(i, L)]
        yv = y_v[pl.ds(i, L)]
        al = jnp.abs(lv)
        t = jnp.exp(-al)
        w = 1.0 / (1.0 + t)
        p = jnp.where(lv >= 0.0, w, t * w)
        p_v[pl.ds(i, L)] = p
        z = t / (2.0 + t)
        sp = jnp.maximum(-lv, 0.0) + _log1p_poly_z(z)
        return b_acc + sp * (1.0 + 2.0 * yv) + (1.0 - yv) * lv

    b_acc = plsc.parallel_loop(0, C2, step=L, unroll=4, carry=zeros)(body)
    pltpu.sync_copy(p_v, p_hbm.at[pl.ds(base, C2)])
    st_v[...] = b_acc
    pltpu.sync_copy(st_v, out_hbm.at[pl.ds(wid * L, L)])


_sc_bce = pl.kernel(
    _sc_bce_body,
    out_type=(
        jax.ShapeDtypeStruct((N_EDGES,), jnp.float32),
        jax.ShapeDtypeStruct((NC * NS * L,), jnp.float32),
    ),
    mesh=plsc.VectorSubcoreMesh(core_axis_name="c", subcore_axis_name="s"),
    scratch_types=[
        pltpu.VMEM((C2,), jnp.float32),
        pltpu.VMEM((C2,), jnp.float32),
        pltpu.VMEM((C2,), jnp.float32),
        pltpu.VMEM((L,), jnp.float32),
        pltpu.SemaphoreType.DMA,
        pltpu.SemaphoreType.DMA,
    ],
)


def _tc_rank_body(d_ref, p_ref, b_ref, out_ref):
    d = d_ref[...]
    maxd = jnp.max(d)
    mind = jnp.min(d)
    m = maxd + 1e-8
    min_s = WEIGHT_DISTANCE * (1.0 - maxd / m)
    max_s = WEIGHT_DISTANCE * (1.0 - mind / m)
    den = max_s - min_s + 1e-8
    coef = (WEIGHT_DISTANCE / m) / den
    inv = 1.0 - coef * (maxd - d)
    r = p_ref[...] - inv
    rank_mean = jnp.sum(r * r) * (1.0 / N_EDGES)
    bce_mean = jnp.sum(b_ref[...]) * (1.0 / N_EDGES)
    total = (1.0 - ALPHA) * bce_mean + ALPHA * rank_mean
    out_ref[...] = total.reshape(1, 1)


_tc_rank = pl.pallas_call(
    _tc_rank_body,
    out_shape=jax.ShapeDtypeStruct((1, 1), jnp.float32),
    in_specs=[
        pl.BlockSpec(memory_space=pltpu.VMEM),
        pl.BlockSpec(memory_space=pltpu.VMEM),
        pl.BlockSpec(memory_space=pltpu.VMEM),
    ],
    out_specs=pl.BlockSpec(memory_space=pltpu.VMEM),
)


def kernel(logits, x, edge_index, edge_attr, y):
    del x, edge_index
    p_flat, bce_out = _sc_bce(logits, y)
    d = jnp.reshape(edge_attr, (N_EDGES,))
    d2 = jnp.reshape(d, (2500, 128))
    p2 = jnp.reshape(p_flat, (2500, 128))
    b2 = jnp.reshape(bce_out, (4, 128))
    total = _tc_rank(d2, p2, b2)
    return total[0, 0]
